# Initial kernel scaffold; baseline (speedup 1.0000x reference)
import functools

import jax
import jax.numpy as jnp
from jax import lax
from jax.experimental import pallas as pl
from jax.experimental.pallas import tpu as pltpu
from jax.experimental.pallas import tpu_sc as plsc

N = 10000
E = 320000
D = 128

NP = 10240
NC = 2
NS = 16
NW = NC * NS
CHUNK = 128
QUOTA = ((E + NW * CHUNK - 1) // (NW * CHUNK)) * CHUNK
EPAD = QUOTA * NW
STRIPE = NP // NS


def _tables_body(emb_ref, w1_ref, w2_ref, b_ref, a_ref, s_ref):
    b = b_ref[0]
    emb = emb_ref[...]
    a_ref[...] = jnp.sum(emb * w1_ref[...], axis=1) + b
    s_ref[...] = jnp.sum(emb * w2_ref[...], axis=1)


def _make_tables(emb_pad, w1, w2, b):
    blk = 1280
    grid = (NP // blk,)
    return pl.pallas_call(
        _tables_body,
        grid=grid,
        in_specs=[
            pl.BlockSpec((blk, D), lambda i: (i, 0)),
            pl.BlockSpec((1, D), lambda i: (0, 0)),
            pl.BlockSpec((1, D), lambda i: (0, 0)),
            pl.BlockSpec(memory_space=pltpu.SMEM),
        ],
        out_specs=[
            pl.BlockSpec((blk,), lambda i: (i,)),
            pl.BlockSpec((blk,), lambda i: (i,)),
        ],
        out_shape=[
            jax.ShapeDtypeStruct((NP,), jnp.float32),
            jax.ShapeDtypeStruct((NP,), jnp.float32),
        ],
    )(emb_pad, w1, w2, b)


def _sum_body(p0_ref, p1_ref, o_ref):
    o_ref[...] = p0_ref[...] + p1_ref[...]


def _sum_partials(zparts):
    blk = 1280
    return pl.pallas_call(
        _sum_body,
        grid=(NP // blk,),
        in_specs=[
            pl.BlockSpec((blk, D), lambda i: (i, 0)),
            pl.BlockSpec((blk, D), lambda i: (i, 0)),
        ],
        out_specs=pl.BlockSpec((blk, D), lambda i: (i, 0)),
        out_shape=jax.ShapeDtypeStruct((NP, D), jnp.float32),
    )(zparts[0], zparts[1])


def _sc_body(emb_hbm, a_hbm, s_hbm, d_hbm, src_hbm, dst_hbm, zout_hbm,
             a_t, s_t, d_t, src_v, dst_v, rows_v, evals_v, z_sh, sem):
    cid = lax.axis_index("c")
    sid = lax.axis_index("s")
    wid = cid * NS + sid

    pltpu.sync_copy(a_hbm, a_t)
    pltpu.sync_copy(s_hbm, s_t)
    pltpu.sync_copy(d_hbm, d_t)

    zero = jnp.zeros((16,), jnp.float32)

    def zero_row(r, carry):
        for k in range(D // 16):
            rows_v[r, pl.ds(k * 16, 16)] = zero
        return carry

    lax.fori_loop(0, CHUNK, zero_row, 0)
    for i in range(STRIPE // CHUNK):
        pltpu.sync_copy(rows_v, z_sh.at[pl.ds(sid * STRIPE + i * CHUNK, CHUNK)])
    plsc.subcore_barrier()

    base0 = wid * QUOTA

    def step(c, carry):
        base = base0 + c * CHUNK
        pltpu.sync_copy(src_hbm.at[pl.ds(base, CHUNK)], src_v)
        pltpu.sync_copy(dst_hbm.at[pl.ds(base, CHUNK)], dst_v)
        gather = pltpu.async_copy(emb_hbm.at[src_v], rows_v, sem)
        for g in range(CHUNK // 16):
            sv = src_v[pl.ds(g * 16, 16)]
            dv = dst_v[pl.ds(g * 16, 16)]
            t = plsc.load_gather(a_t, [dv]) + plsc.load_gather(s_t, [sv])
            g_val = 1.0 - 2.0 / (jnp.exp(2.0 * t) + 1.0)
            e = g_val * plsc.load_gather(d_t, [dv]) * plsc.load_gather(d_t, [sv])
            evals_v[pl.ds(g * 16, 16)] = e
        gather.wait()

        def scale_row(j, inner):
            e_spl = plsc.load_gather(evals_v, [jnp.full((16,), j, jnp.int32)])
            for k in range(D // 16):
                rows_v[j, pl.ds(k * 16, 16)] = rows_v[j, pl.ds(k * 16, 16)] * e_spl
            return inner

        lax.fori_loop(0, CHUNK, scale_row, 0)
        pltpu.sync_copy(rows_v, z_sh.at[dst_v], add=True)
        return carry

    lax.fori_loop(0, QUOTA // CHUNK, step, 0)

    plsc.subcore_barrier()
    pltpu.sync_copy(
        z_sh.at[pl.ds(sid * STRIPE, STRIPE)],
        zout_hbm.at[pl.ds(cid * NP + sid * STRIPE, STRIPE)],
    )


@functools.partial(
    pl.kernel,
    mesh=plsc.VectorSubcoreMesh(core_axis_name="c", subcore_axis_name="s"),
    out_type=jax.ShapeDtypeStruct((NC * NP, D), jnp.float32),
    scratch_types=[
        pltpu.VMEM((NP,), jnp.float32),
        pltpu.VMEM((NP,), jnp.float32),
        pltpu.VMEM((NP,), jnp.float32),
        pltpu.VMEM((CHUNK,), jnp.int32),
        pltpu.VMEM((CHUNK,), jnp.int32),
        pltpu.VMEM((CHUNK, D), jnp.float32),
        pltpu.VMEM((CHUNK,), jnp.float32),
        pltpu.VMEM_SHARED((NP, D), jnp.float32),
        pltpu.SemaphoreType.DMA,
    ],
)
def _sc_aggregate(emb_hbm, a_hbm, s_hbm, d_hbm, src_hbm, dst_hbm, zout_hbm,
                  a_t, s_t, d_t, src_v, dst_v, rows_v, evals_v, z_sh, sem):
    _sc_body(emb_hbm, a_hbm, s_hbm, d_hbm, src_hbm, dst_hbm, zout_hbm,
             a_t, s_t, d_t, src_v, dst_v, rows_v, evals_v, z_sh, sem)


@jax.jit
def kernel(emb, d, edge_index, gate_W, gate_b):
    emb_pad = jnp.zeros((NP, D), jnp.float32).at[:N].set(emb)
    d_pad = jnp.zeros((NP,), jnp.float32).at[:N].set(d)
    w1 = gate_W[:, :D]
    w2 = gate_W[:, D:]
    a_tbl, s_tbl = _make_tables(emb_pad, w1, w2, gate_b)
    pad = jnp.full((EPAD - E,), N, jnp.int32)
    src = jnp.concatenate([edge_index[0], pad])
    dst = jnp.concatenate([edge_index[1], pad])
    zparts = _sc_aggregate(emb_pad, a_tbl, s_tbl, d_pad, src, dst)
    z = _sum_partials(zparts.reshape(NC, NP, D))
    return z[:N]


# speedup vs baseline: 11.7041x; 11.7041x over previous
import functools

import jax
import jax.numpy as jnp
from jax import lax
from jax.experimental import pallas as pl
from jax.experimental.pallas import tpu as pltpu
from jax.experimental.pallas import tpu_sc as plsc

N = 10000
E = 320000
D = 128

NP = 10240
NC = 2
NS = 16
NW = NC * NS
CHUNK = 128
QUOTA = ((E + NW * CHUNK - 1) // (NW * CHUNK)) * CHUNK
EPAD = QUOTA * NW
STRIPE = NP // NS


def _tables_body(emb_ref, w1_ref, w2_ref, b_ref, a_ref, s_ref):
    b = b_ref[0]
    emb = emb_ref[...]
    a_ref[...] = jnp.sum(emb * w1_ref[...], axis=1) + b
    s_ref[...] = jnp.sum(emb * w2_ref[...], axis=1)


def _make_tables(emb_pad, w1, w2, b):
    blk = 2048
    grid = (NP // blk,)
    return pl.pallas_call(
        _tables_body,
        grid=grid,
        in_specs=[
            pl.BlockSpec((blk, D), lambda i: (i, 0)),
            pl.BlockSpec((1, D), lambda i: (0, 0)),
            pl.BlockSpec((1, D), lambda i: (0, 0)),
            pl.BlockSpec(memory_space=pltpu.SMEM),
        ],
        out_specs=[
            pl.BlockSpec((blk,), lambda i: (i,)),
            pl.BlockSpec((blk,), lambda i: (i,)),
        ],
        out_shape=[
            jax.ShapeDtypeStruct((NP,), jnp.float32),
            jax.ShapeDtypeStruct((NP,), jnp.float32),
        ],
    )(emb_pad, w1, w2, b)


def _sum_body(p0_ref, p1_ref, o_ref):
    o_ref[...] = p0_ref[...] + p1_ref[...]


def _sum_partials(zparts):
    blk = 1280
    return pl.pallas_call(
        _sum_body,
        grid=(NP // blk,),
        in_specs=[
            pl.BlockSpec((blk, D), lambda i: (i, 0)),
            pl.BlockSpec((blk, D), lambda i: (i, 0)),
        ],
        out_specs=pl.BlockSpec((blk, D), lambda i: (i, 0)),
        out_shape=jax.ShapeDtypeStruct((NP, D), jnp.float32),
    )(zparts[0], zparts[1])


def _sc_body(emb_hbm, a_hbm, s_hbm, d_hbm, src_hbm, dst_hbm, zout_hbm,
             a_t, s_t, d_t, src_v, dst_v, rows_v, evals_v, z_sh, sem):
    cid = lax.axis_index("c")
    sid = lax.axis_index("s")
    wid = cid * NS + sid

    pltpu.sync_copy(a_hbm, a_t)
    pltpu.sync_copy(s_hbm, s_t)
    pltpu.sync_copy(d_hbm, d_t)

    zero = jnp.zeros((16,), jnp.float32)

    def zero_row(r, carry):
        for k in range(D // 16):
            rows_v[r, pl.ds(k * 16, 16)] = zero
        return carry

    lax.fori_loop(0, CHUNK, zero_row, 0)
    for i in range(STRIPE // CHUNK):
        pltpu.sync_copy(rows_v, z_sh.at[pl.ds(sid * STRIPE + i * CHUNK, CHUNK)])
    plsc.subcore_barrier()

    base0 = wid * QUOTA

    def step(c, carry):
        base = base0 + c * CHUNK
        pltpu.sync_copy(src_hbm.at[pl.ds(base, CHUNK)], src_v)
        pltpu.sync_copy(dst_hbm.at[pl.ds(base, CHUNK)], dst_v)
        gather = pltpu.async_copy(emb_hbm.at[src_v], rows_v, sem)
        for g in range(CHUNK // 16):
            sv = src_v[pl.ds(g * 16, 16)]
            dv = dst_v[pl.ds(g * 16, 16)]
            t = plsc.load_gather(a_t, [dv]) + plsc.load_gather(s_t, [sv])
            g_val = 1.0 - 2.0 / (jnp.exp(2.0 * t) + 1.0)
            e = g_val * plsc.load_gather(d_t, [dv]) * plsc.load_gather(d_t, [sv])
            evals_v[pl.ds(g * 16, 16)] = e
        gather.wait()

        def scale_row(j, inner):
            e_spl = plsc.load_gather(evals_v, [jnp.full((16,), j, jnp.int32)])
            for k in range(D // 16):
                rows_v[j, pl.ds(k * 16, 16)] = rows_v[j, pl.ds(k * 16, 16)] * e_spl
            return inner

        lax.fori_loop(0, CHUNK, scale_row, 0)
        pltpu.sync_copy(rows_v, z_sh.at[dst_v], add=True)
        return carry

    lax.fori_loop(0, QUOTA // CHUNK, step, 0)

    plsc.subcore_barrier()
    pltpu.sync_copy(
        z_sh.at[pl.ds(sid * STRIPE, STRIPE)],
        zout_hbm.at[pl.ds(cid * NP + sid * STRIPE, STRIPE)],
    )


@functools.partial(
    pl.kernel,
    mesh=plsc.VectorSubcoreMesh(core_axis_name="c", subcore_axis_name="s"),
    out_type=jax.ShapeDtypeStruct((NC * NP, D), jnp.float32),
    compiler_params=pltpu.CompilerParams(needs_layout_passes=False),
    scratch_types=[
        pltpu.VMEM((NP,), jnp.float32),
        pltpu.VMEM((NP,), jnp.float32),
        pltpu.VMEM((NP,), jnp.float32),
        pltpu.VMEM((CHUNK,), jnp.int32),
        pltpu.VMEM((CHUNK,), jnp.int32),
        pltpu.VMEM((CHUNK, D), jnp.float32),
        pltpu.VMEM((CHUNK,), jnp.float32),
        pltpu.VMEM_SHARED((NP, D), jnp.float32),
        pltpu.SemaphoreType.DMA,
    ],
)
def _sc_aggregate(emb_hbm, a_hbm, s_hbm, d_hbm, src_hbm, dst_hbm, zout_hbm,
                  a_t, s_t, d_t, src_v, dst_v, rows_v, evals_v, z_sh, sem):
    _sc_body(emb_hbm, a_hbm, s_hbm, d_hbm, src_hbm, dst_hbm, zout_hbm,
             a_t, s_t, d_t, src_v, dst_v, rows_v, evals_v, z_sh, sem)


@jax.jit
def kernel(emb, d, edge_index, gate_W, gate_b):
    emb_pad = jnp.zeros((NP, D), jnp.float32).at[:N].set(emb)
    d_pad = jnp.zeros((NP,), jnp.float32).at[:N].set(d)
    w1 = gate_W[:, :D]
    w2 = gate_W[:, D:]
    a_tbl, s_tbl = _make_tables(emb_pad, w1, w2, gate_b)
    pad = jnp.full((EPAD - E,), N, jnp.int32)
    src = jnp.concatenate([edge_index[0], pad])
    dst = jnp.concatenate([edge_index[1], pad])
    zparts = _sc_aggregate(emb_pad, a_tbl, s_tbl, d_pad, src, dst)
    z = _sum_partials(zparts.reshape(NC, NP, D))
    return z[:N]
